# baseline (speedup 1.0000x reference)
import functools

import jax
import jax.numpy as jnp
from jax import lax
from jax.experimental import pallas as pl
from jax.experimental.pallas import tpu as pltpu
from jax.experimental.pallas import tpu_sc as plsc

BATCH = 16384
EMB = 64

_NC = 2
_NS = 16
_NW = _NC * _NS
_BPW = BATCH // _NW
_CH = 128
_NCH = _BPW // _CH


def _sc_gather(table, idx):
    mesh = plsc.VectorSubcoreMesh(core_axis_name="c", subcore_axis_name="s")

    @functools.partial(
        pl.kernel,
        mesh=mesh,
        out_type=jax.ShapeDtypeStruct((BATCH, EMB), jnp.float32),
        scratch_types=[
            pltpu.VMEM((_NCH, _CH), jnp.int32),
            pltpu.VMEM((_BPW, EMB), jnp.float32),
            pltpu.SemaphoreType.DMA,
        ],
        compiler_params=pltpu.CompilerParams(use_tc_tiling_on_sc=False),
    )
    def gather_kernel(table_hbm, idx_hbm, out_hbm, idx_v, rows_v, sem):
        wid = lax.axis_index("s") * _NC + lax.axis_index("c")
        base = wid * _BPW
        for c in range(_NCH):
            pltpu.sync_copy(idx_hbm.at[pl.ds(base + c * _CH, _CH)], idx_v.at[c])
        copies = [
            pltpu.async_copy(
                table_hbm.at[idx_v.at[c]],
                rows_v.at[pl.ds(c * _CH, _CH)],
                sem,
            )
            for c in range(_NCH)
        ]
        for cp in copies:
            cp.wait()
        pltpu.sync_copy(rows_v, out_hbm.at[pl.ds(base, _BPW)])

    return gather_kernel(table, idx)


def _tc_fuse(x, id_emb, W1, b1, W2, b2, Wf1, Wf2, bf):
    BB = 2048

    def body(x_ref, id_ref, w1_ref, b1_ref, w2_ref, b2_ref,
             wf1_ref, wf2_ref, bf_ref, out_ref):
        h = jnp.maximum(
            jnp.dot(x_ref[...], w1_ref[...],
                    preferred_element_type=jnp.float32) + b1_ref[...], 0.0)
        fe = jnp.dot(h, w2_ref[...],
                     preferred_element_type=jnp.float32) + b2_ref[...]
        out_ref[...] = (
            jnp.dot(id_ref[...], wf1_ref[...],
                    preferred_element_type=jnp.float32)
            + jnp.dot(fe, wf2_ref[...], preferred_element_type=jnp.float32)
            + bf_ref[...]
        )

    full = lambda i: (0, 0)
    return pl.pallas_call(
        body,
        grid=(BATCH // BB,),
        in_specs=[
            pl.BlockSpec((BB, 64), lambda i: (i, 0)),
            pl.BlockSpec((BB, 64), lambda i: (i, 0)),
            pl.BlockSpec((64, 64), full),
            pl.BlockSpec((1, 64), full),
            pl.BlockSpec((64, 64), full),
            pl.BlockSpec((1, 64), full),
            pl.BlockSpec((64, 64), full),
            pl.BlockSpec((64, 64), full),
            pl.BlockSpec((1, 64), full),
        ],
        out_specs=pl.BlockSpec((BB, 64), lambda i: (i, 0)),
        out_shape=jax.ShapeDtypeStruct((BATCH, 64), jnp.float32),
    )(x, id_emb, W1, b1, W2, b2, Wf1, Wf2, bf)


def kernel(item_ids, item_features, emb_table, W1, b1, W2, b2, Wf, bf):
    ids = item_ids.astype(jnp.int32)
    id_emb = _sc_gather(emb_table, ids)
    fused = _tc_fuse(
        item_features, id_emb,
        W1, b1.reshape(1, 64), W2, b2.reshape(1, 64),
        Wf[:EMB], Wf[EMB:], bf.reshape(1, 64),
    )
    return fused, id_emb


# speedup vs baseline: 1.5935x; 1.5935x over previous
import functools

import jax
import jax.numpy as jnp
from jax import lax
from jax.experimental import pallas as pl
from jax.experimental.pallas import tpu as pltpu
from jax.experimental.pallas import tpu_sc as plsc

BATCH = 16384
EMB = 64

_NC = 2
_NS = 16
_NW = _NC * _NS
_BPW = BATCH // _NW
_K = 16
_STEPS = _BPW // _K


def _sc_gather(table, idx):
    mesh = plsc.VectorSubcoreMesh(core_axis_name="c", subcore_axis_name="s")

    @functools.partial(
        pl.kernel,
        mesh=mesh,
        out_type=jax.ShapeDtypeStruct((BATCH, EMB), jnp.float32),
        scratch_types=[
            pltpu.VMEM((_BPW,), jnp.int32),
            pltpu.VMEM((_BPW, EMB), jnp.float32),
            pltpu.SemaphoreType.DMA,
        ],
    )
    def gather_kernel(table_hbm, idx_hbm, out_hbm, idx_v, rows_v, sem):
        wid = lax.axis_index("s") * _NC + lax.axis_index("c")
        base = wid * _BPW
        pltpu.sync_copy(idx_hbm.at[pl.ds(base, _BPW)], idx_v)

        def step(s, carry):
            r0 = s * _K
            vec = idx_v[pl.ds(r0, 16)]
            copies = []
            for j in range(_K):
                i = vec[j]
                copies.append(pltpu.async_copy(
                    table_hbm.at[pl.ds(i, 1)],
                    rows_v.at[pl.ds(r0 + j, 1)],
                    sem,
                ))
            for cp in copies:
                cp.wait()
            return carry

        lax.fori_loop(0, _STEPS, step, 0)
        pltpu.sync_copy(rows_v, out_hbm.at[pl.ds(base, _BPW)])

    return gather_kernel(table, idx)


def _tc_fuse(x, id_emb, W1, b1, W2, b2, Wf1, Wf2, bf):
    BB = 2048

    def body(x_ref, id_ref, w1_ref, b1_ref, w2_ref, b2_ref,
             wf1_ref, wf2_ref, bf_ref, out_ref):
        h = jnp.maximum(
            jnp.dot(x_ref[...], w1_ref[...],
                    preferred_element_type=jnp.float32) + b1_ref[...], 0.0)
        fe = jnp.dot(h, w2_ref[...],
                     preferred_element_type=jnp.float32) + b2_ref[...]
        out_ref[...] = (
            jnp.dot(id_ref[...], wf1_ref[...],
                    preferred_element_type=jnp.float32)
            + jnp.dot(fe, wf2_ref[...], preferred_element_type=jnp.float32)
            + bf_ref[...]
        )

    full = lambda i: (0, 0)
    return pl.pallas_call(
        body,
        grid=(BATCH // BB,),
        in_specs=[
            pl.BlockSpec((BB, 64), lambda i: (i, 0)),
            pl.BlockSpec((BB, 64), lambda i: (i, 0)),
            pl.BlockSpec((64, 64), full),
            pl.BlockSpec((1, 64), full),
            pl.BlockSpec((64, 64), full),
            pl.BlockSpec((1, 64), full),
            pl.BlockSpec((64, 64), full),
            pl.BlockSpec((64, 64), full),
            pl.BlockSpec((1, 64), full),
        ],
        out_specs=pl.BlockSpec((BB, 64), lambda i: (i, 0)),
        out_shape=jax.ShapeDtypeStruct((BATCH, 64), jnp.float32),
    )(x, id_emb, W1, b1, W2, b2, Wf1, Wf2, bf)


def kernel(item_ids, item_features, emb_table, W1, b1, W2, b2, Wf, bf):
    ids = item_ids.astype(jnp.int32)
    id_emb = _sc_gather(emb_table, ids)
    fused = _tc_fuse(
        item_features, id_emb,
        W1, b1.reshape(1, 64), W2, b2.reshape(1, 64),
        Wf[:EMB], Wf[EMB:], bf.reshape(1, 64),
    )
    return fused, id_emb


# speedup vs baseline: 1.8103x; 1.1360x over previous
import functools

import jax
import jax.numpy as jnp
from jax import lax
from jax.experimental import pallas as pl
from jax.experimental.pallas import tpu as pltpu
from jax.experimental.pallas import tpu_sc as plsc

BATCH = 16384
EMB = 64
NROWS = 1000001

_C = 2048
_SB = 245
_S = _SB * _C
_LASTB = (NROWS + _C - 1) // _C - 1

_NC = 2
_NS = 16
_NW = _NC * _NS
_BPW = BATCH // _NW
_CH = 128
_NCH = _BPW // _CH


def _tc_pack(table_t):
    def body(a_ref, b_ref, out_ref):
        out_ref[...] = jnp.concatenate([a_ref[...].T, b_ref[...].T], axis=1)

    return pl.pallas_call(
        body,
        grid=(_SB,),
        in_specs=[
            pl.BlockSpec((64, _C), lambda i: (0, i)),
            pl.BlockSpec((64, _C), lambda i: (0, jnp.minimum(_SB + i, _LASTB))),
        ],
        out_specs=pl.BlockSpec((_C, 128), lambda i: (i, 0)),
        out_shape=jax.ShapeDtypeStruct((_S, 128), jnp.float32),
    )(table_t, table_t)


def _sc_gather(packed, idx):
    mesh = plsc.VectorSubcoreMesh(core_axis_name="c", subcore_axis_name="s")

    @functools.partial(
        pl.kernel,
        mesh=mesh,
        out_type=jax.ShapeDtypeStruct((BATCH, 128), jnp.float32),
        scratch_types=[
            pltpu.VMEM((_NCH, _CH), jnp.int32),
            pltpu.VMEM((_BPW, 128), jnp.float32),
            pltpu.SemaphoreType.DMA,
        ],
    )
    def gather_kernel(packed_hbm, idx_hbm, out_hbm, q_v, rows_v, sem):
        wid = lax.axis_index("s") * _NC + lax.axis_index("c")
        base = wid * _BPW
        for c in range(_NCH):
            pltpu.sync_copy(idx_hbm.at[pl.ds(base + c * _CH, _CH)],
                            q_v.at[c])
        for v in range(_BPW // 16):
            c, o = divmod(v * 16, _CH)
            vec = q_v[c, pl.ds(o, 16)]
            q_v[c, pl.ds(o, 16)] = jnp.where(vec >= _S, vec - _S, vec)
        copies = [
            pltpu.async_copy(
                packed_hbm.at[q_v.at[c]],
                rows_v.at[pl.ds(c * _CH, _CH)],
                sem,
            )
            for c in range(_NCH)
        ]
        for cp in copies:
            cp.wait()
        pltpu.sync_copy(rows_v, out_hbm.at[pl.ds(base, _BPW)])

    return gather_kernel(packed, idx)


def _tc_fuse(x_t, gathered, ids3, W1t, b1c, W2t, b2c, Wf1t, Wf2t, bfc):
    BB = 2048

    def body(xt_ref, g_ref, ids_ref, w1_ref, b1_ref, w2_ref, b2_ref,
             wf1_ref, wf2_ref, bf_ref, out_ref, id_out_ref):
        g = g_ref[...]
        hi = ids_ref[0, :, :] >= _S
        idt = jnp.where(hi, g[:, 64:].T, g[:, :64].T)
        id_out_ref[...] = idt
        h = jnp.maximum(
            jnp.dot(w1_ref[...], xt_ref[...],
                    preferred_element_type=jnp.float32) + b1_ref[...], 0.0)
        fe = jnp.dot(w2_ref[...], h,
                     preferred_element_type=jnp.float32) + b2_ref[...]
        out_ref[...] = (
            jnp.dot(wf1_ref[...], idt, preferred_element_type=jnp.float32)
            + jnp.dot(wf2_ref[...], fe, preferred_element_type=jnp.float32)
            + bf_ref[...]
        )

    full = lambda i: (0, 0)
    return pl.pallas_call(
        body,
        grid=(BATCH // BB,),
        in_specs=[
            pl.BlockSpec((64, BB), lambda i: (0, i)),
            pl.BlockSpec((BB, 128), lambda i: (i, 0)),
            pl.BlockSpec((1, 1, BB), lambda i: (i, 0, 0)),
            pl.BlockSpec((64, 64), full),
            pl.BlockSpec((64, 1), full),
            pl.BlockSpec((64, 64), full),
            pl.BlockSpec((64, 1), full),
            pl.BlockSpec((64, 64), full),
            pl.BlockSpec((64, 64), full),
            pl.BlockSpec((64, 1), full),
        ],
        out_specs=[
            pl.BlockSpec((64, BB), lambda i: (0, i)),
            pl.BlockSpec((64, BB), lambda i: (0, i)),
        ],
        out_shape=[
            jax.ShapeDtypeStruct((EMB, BATCH), jnp.float32),
            jax.ShapeDtypeStruct((EMB, BATCH), jnp.float32),
        ],
    )(x_t, gathered, ids3, W1t, b1c, W2t, b2c, Wf1t, Wf2t, bfc)


def kernel(item_ids, item_features, emb_table, W1, b1, W2, b2, Wf, bf):
    ids = item_ids.astype(jnp.int32)
    table_t = emb_table.T
    x_t = item_features.T
    packed = _tc_pack(table_t)
    gathered = _sc_gather(packed, ids)
    out_t, id_emb_t = _tc_fuse(
        x_t, gathered, ids.reshape(8, 1, BATCH // 8),
        W1.T, b1.reshape(64, 1), W2.T, b2.reshape(64, 1),
        Wf[:EMB].T, Wf[EMB:].T, bf.reshape(64, 1),
    )
    return out_t.T, id_emb_t.T


# speedup vs baseline: 1.9759x; 1.0915x over previous
import functools

import jax
import jax.numpy as jnp
from jax import lax
from jax.experimental import pallas as pl
from jax.experimental.pallas import tpu as pltpu
from jax.experimental.pallas import tpu_sc as plsc

BATCH = 16384
EMB = 64
NROWS = 1000001

_C = 2048
_SB = 245
_S = _SB * _C
_LASTB = (NROWS + _C - 1) // _C - 1

_NC = 2
_NS = 16
_NW = _NC * _NS
_BPW = BATCH // _NW
_CH = 128
_NCH = _BPW // _CH


def _tc_pack(table_t):
    def body(a_ref, b_ref, out_ref):
        eye = (jax.lax.broadcasted_iota(jnp.int32, (64, 64), 0)
               == jax.lax.broadcasted_iota(jnp.int32, (64, 64), 1)
               ).astype(jnp.float32)
        dn = (((0,), (0,)), ((), ()))
        ta = jax.lax.dot_general(a_ref[...], eye, dn,
                                 preferred_element_type=jnp.float32)
        tb = jax.lax.dot_general(b_ref[...], eye, dn,
                                 preferred_element_type=jnp.float32)
        out_ref[...] = jnp.concatenate([ta, tb], axis=1).astype(jnp.bfloat16)

    return pl.pallas_call(
        body,
        grid=(_SB,),
        in_specs=[
            pl.BlockSpec((64, _C), lambda i: (0, i)),
            pl.BlockSpec((64, _C), lambda i: (0, jnp.minimum(_SB + i, _LASTB))),
        ],
        out_specs=pl.BlockSpec((_C, 128), lambda i: (i, 0)),
        out_shape=jax.ShapeDtypeStruct((_S, 128), jnp.bfloat16),
    )(table_t, table_t)


def _sc_gather(packed, idx):
    mesh = plsc.VectorSubcoreMesh(core_axis_name="c", subcore_axis_name="s")

    @functools.partial(
        pl.kernel,
        mesh=mesh,
        out_type=jax.ShapeDtypeStruct((BATCH, 128), jnp.int32),
        scratch_types=[
            pltpu.VMEM((_NCH, _CH), jnp.int32),
            pltpu.VMEM((_BPW, 128), jnp.int32),
            pltpu.SemaphoreType.DMA,
        ],
    )
    def gather_kernel(packed_hbm, idx_hbm, out_hbm, q_v, rows_v, sem):
        p32 = packed_hbm.bitcast(jnp.int32)
        wid = lax.axis_index("s") * _NC + lax.axis_index("c")
        base = wid * _BPW
        for c in range(_NCH):
            pltpu.sync_copy(idx_hbm.at[pl.ds(base + c * _CH, _CH)],
                            q_v.at[c])
        for v in range(_BPW // 16):
            c, o = divmod(v * 16, _CH)
            vec = q_v[c, pl.ds(o, 16)]
            q = jnp.where(vec >= _S, vec - _S, vec)
            q_v[c, pl.ds(o, 16)] = q >> 1
        copies = [
            pltpu.async_copy(
                p32.at[q_v.at[c]],
                rows_v.at[pl.ds(c * _CH, _CH)],
                sem,
            )
            for c in range(_NCH)
        ]
        for cp in copies:
            cp.wait()
        pltpu.sync_copy(rows_v, out_hbm.at[pl.ds(base, _BPW)])

    return gather_kernel(packed, idx)


def _tc_fuse(x_t, gathered, ids_col, W1t, b1c, W2t, b2c, Wf1t, Wf2t, bfc):
    BB = 2048

    def body(xt_ref, g_ref, ids_ref, w1_ref, b1_ref, w2_ref, b2_ref,
             wf1_ref, wf2_ref, bf_ref, out_ref, id_out_ref):
        g = g_ref[...]
        ids = ids_ref[...]
        hi = ids >= _S
        q = jnp.where(hi, ids - _S, ids)
        odd = (q & 1) == 1
        sel = jnp.where(hi, g[:, 64:], g[:, :64])
        bits = jnp.where(odd, sel & jnp.int32(-65536), sel << 16)
        vals = jax.lax.bitcast_convert_type(bits, jnp.float32)
        idt = vals.T
        id_out_ref[...] = idt
        h = jnp.maximum(
            jnp.dot(w1_ref[...], xt_ref[...],
                    preferred_element_type=jnp.float32) + b1_ref[...], 0.0)
        fe = jnp.dot(w2_ref[...], h,
                     preferred_element_type=jnp.float32) + b2_ref[...]
        out_ref[...] = (
            jnp.dot(wf1_ref[...], idt, preferred_element_type=jnp.float32)
            + jnp.dot(wf2_ref[...], fe, preferred_element_type=jnp.float32)
            + bf_ref[...]
        )

    full = lambda i: (0, 0)
    return pl.pallas_call(
        body,
        grid=(BATCH // BB,),
        in_specs=[
            pl.BlockSpec((64, BB), lambda i: (0, i)),
            pl.BlockSpec((BB, 128), lambda i: (i, 0)),
            pl.BlockSpec((BB, 1), lambda i: (i, 0)),
            pl.BlockSpec((64, 64), full),
            pl.BlockSpec((64, 1), full),
            pl.BlockSpec((64, 64), full),
            pl.BlockSpec((64, 1), full),
            pl.BlockSpec((64, 64), full),
            pl.BlockSpec((64, 64), full),
            pl.BlockSpec((64, 1), full),
        ],
        out_specs=[
            pl.BlockSpec((64, BB), lambda i: (0, i)),
            pl.BlockSpec((64, BB), lambda i: (0, i)),
        ],
        out_shape=[
            jax.ShapeDtypeStruct((EMB, BATCH), jnp.float32),
            jax.ShapeDtypeStruct((EMB, BATCH), jnp.float32),
        ],
    )(x_t, gathered, ids_col, W1t, b1c, W2t, b2c, Wf1t, Wf2t, bfc)


def kernel(item_ids, item_features, emb_table, W1, b1, W2, b2, Wf, bf):
    ids = item_ids.astype(jnp.int32)
    table_t = emb_table.T
    x_t = item_features.T
    packed = _tc_pack(table_t)
    gathered = _sc_gather(packed, ids)
    out_t, id_emb_t = _tc_fuse(
        x_t, gathered, ids.reshape(BATCH, 1),
        W1.T, b1.reshape(64, 1), W2.T, b2.reshape(64, 1),
        Wf[:EMB].T, Wf[EMB:].T, bf.reshape(64, 1),
    )
    return out_t.T, id_emb_t.T


# speedup vs baseline: 2.5007x; 1.2656x over previous
import functools

import jax
import jax.numpy as jnp
from jax import lax
from jax.experimental import pallas as pl
from jax.experimental.pallas import tpu as pltpu
from jax.experimental.pallas import tpu_sc as plsc

BATCH = 16384
EMB = 64
NROWS = 1000001

_C = 4096
_SB = 123
_S = _SB * _C
_LASTB = (NROWS + _C - 1) // _C - 1

_NC = 2
_NS = 16
_NW = _NC * _NS
_BPW = BATCH // _NW
_CH = 128
_NCH = _BPW // _CH


def _tc_pack(table_t):
    def body(a_ref, b_ref, out_ref):
        eye = (jax.lax.broadcasted_iota(jnp.int32, (64, 64), 0)
               == jax.lax.broadcasted_iota(jnp.int32, (64, 64), 1)
               ).astype(jnp.float32)
        dn = (((0,), (0,)), ((), ()))
        ta = jax.lax.dot_general(a_ref[...], eye, dn,
                                 preferred_element_type=jnp.float32)
        tb = jax.lax.dot_general(b_ref[...], eye, dn,
                                 preferred_element_type=jnp.float32)
        out_ref[...] = jnp.concatenate([ta, tb], axis=1).astype(jnp.bfloat16)

    return pl.pallas_call(
        body,
        grid=(_SB,),
        in_specs=[
            pl.BlockSpec((64, _C), lambda i: (0, i)),
            pl.BlockSpec((64, _C), lambda i: (0, jnp.minimum(_SB + i, _LASTB))),
        ],
        out_specs=pl.BlockSpec((_C, 128), lambda i: (i, 0)),
        out_shape=jax.ShapeDtypeStruct((_S, 128), jnp.bfloat16),
    )(table_t, table_t)


def _sc_gather(packed, idx):
    mesh = plsc.VectorSubcoreMesh(core_axis_name="c", subcore_axis_name="s")

    @functools.partial(
        pl.kernel,
        mesh=mesh,
        out_type=jax.ShapeDtypeStruct((BATCH, 128), jnp.int32),
        scratch_types=[
            pltpu.VMEM((_NCH, _CH), jnp.int32),
            pltpu.VMEM((_BPW, 128), jnp.int32),
            pltpu.SemaphoreType.DMA,
        ],
    )
    def gather_kernel(packed_hbm, idx_hbm, out_hbm, q_v, rows_v, sem):
        p32 = packed_hbm.bitcast(jnp.int32)
        wid = lax.axis_index("s") * _NC + lax.axis_index("c")
        base = wid * _BPW
        for c in range(_NCH):
            pltpu.sync_copy(idx_hbm.at[pl.ds(base + c * _CH, _CH)],
                            q_v.at[c])
        for v in range(_BPW // 16):
            c, o = divmod(v * 16, _CH)
            vec = q_v[c, pl.ds(o, 16)]
            q = jnp.where(vec >= _S, vec - _S, vec)
            q_v[c, pl.ds(o, 16)] = q >> 1
        copies = [
            pltpu.async_copy(
                p32.at[q_v.at[c]],
                rows_v.at[pl.ds(c * _CH, _CH)],
                sem,
            )
            for c in range(_NCH)
        ]
        for cp in copies:
            cp.wait()
        pltpu.sync_copy(rows_v, out_hbm.at[pl.ds(base, _BPW)])

    return gather_kernel(packed, idx)


def _tc_fuse(x_t, gathered, ids_col, W1t, b1c, W2t, b2c, Wf1t, Wf2t, bfc):
    BB = 2048

    def body(xt_ref, g_ref, ids_ref, w1_ref, b1_ref, w2_ref, b2_ref,
             wf1_ref, wf2_ref, bf_ref, out_ref, id_out_ref):
        g = g_ref[...]
        ids = ids_ref[...]
        hi = ids >= _S
        q = jnp.where(hi, ids - _S, ids)
        odd = (q & 1) == 1
        sel = jnp.where(hi, g[:, 64:], g[:, :64])
        bits = jnp.where(odd, sel & jnp.int32(-65536), sel << 16)
        vals = jax.lax.bitcast_convert_type(bits, jnp.float32)
        idt = vals.T
        id_out_ref[...] = idt
        h = jnp.maximum(
            jnp.dot(w1_ref[...], xt_ref[...],
                    preferred_element_type=jnp.float32) + b1_ref[...], 0.0)
        fe = jnp.dot(w2_ref[...], h,
                     preferred_element_type=jnp.float32) + b2_ref[...]
        out_ref[...] = (
            jnp.dot(wf1_ref[...], idt, preferred_element_type=jnp.float32)
            + jnp.dot(wf2_ref[...], fe, preferred_element_type=jnp.float32)
            + bf_ref[...]
        )

    full = lambda i: (0, 0)
    return pl.pallas_call(
        body,
        grid=(BATCH // BB,),
        in_specs=[
            pl.BlockSpec((64, BB), lambda i: (0, i)),
            pl.BlockSpec((BB, 128), lambda i: (i, 0)),
            pl.BlockSpec((BB, 1), lambda i: (i, 0)),
            pl.BlockSpec((64, 64), full),
            pl.BlockSpec((64, 1), full),
            pl.BlockSpec((64, 64), full),
            pl.BlockSpec((64, 1), full),
            pl.BlockSpec((64, 64), full),
            pl.BlockSpec((64, 64), full),
            pl.BlockSpec((64, 1), full),
        ],
        out_specs=[
            pl.BlockSpec((64, BB), lambda i: (0, i)),
            pl.BlockSpec((64, BB), lambda i: (0, i)),
        ],
        out_shape=[
            jax.ShapeDtypeStruct((EMB, BATCH), jnp.float32),
            jax.ShapeDtypeStruct((EMB, BATCH), jnp.float32),
        ],
    )(x_t, gathered, ids_col, W1t, b1c, W2t, b2c, Wf1t, Wf2t, bfc)


def kernel(item_ids, item_features, emb_table, W1, b1, W2, b2, Wf, bf):
    ids = item_ids.astype(jnp.int32)
    table_t = emb_table.T
    x_t = item_features.T
    packed = _tc_pack(table_t)
    gathered = _sc_gather(packed, ids)
    out_t, id_emb_t = _tc_fuse(
        x_t, gathered, ids.reshape(BATCH, 1),
        W1.T, b1.reshape(64, 1), W2.T, b2.reshape(64, 1),
        Wf[:EMB].T, Wf[EMB:].T, bf.reshape(64, 1),
    )
    return out_t.T, id_emb_t.T


# speedup vs baseline: 2.8552x; 1.1418x over previous
import functools

import jax
import jax.numpy as jnp
from jax import lax
from jax.experimental import pallas as pl
from jax.experimental.pallas import tpu as pltpu
from jax.experimental.pallas import tpu_sc as plsc

BATCH = 16384
EMB = 64
NROWS = 1000001

_C = 8192
_SB = 62
_S = _SB * _C
_LASTB = (NROWS + _C - 1) // _C - 1

_NC = 2
_NS = 16
_NW = _NC * _NS
_BPW = BATCH // _NW
_CH = 128
_NCH = _BPW // _CH


def _tc_pack(table_t):
    def body(a_ref, b_ref, out_ref):
        eye = (jax.lax.broadcasted_iota(jnp.int32, (64, 64), 0)
               == jax.lax.broadcasted_iota(jnp.int32, (64, 64), 1)
               ).astype(jnp.float32)
        dn = (((0,), (0,)), ((), ()))
        ta = jax.lax.dot_general(a_ref[...], eye, dn,
                                 preferred_element_type=jnp.float32)
        tb = jax.lax.dot_general(b_ref[...], eye, dn,
                                 preferred_element_type=jnp.float32)
        out_ref[...] = jnp.concatenate([ta, tb], axis=1).astype(jnp.bfloat16)

    return pl.pallas_call(
        body,
        grid=(_SB,),
        in_specs=[
            pl.BlockSpec((64, _C), lambda i: (0, i)),
            pl.BlockSpec((64, _C), lambda i: (0, jnp.minimum(_SB + i, _LASTB))),
        ],
        out_specs=pl.BlockSpec((_C, 128), lambda i: (i, 0)),
        out_shape=jax.ShapeDtypeStruct((_S, 128), jnp.bfloat16),
    )(table_t, table_t)


def _sc_gather(packed, idx):
    mesh = plsc.VectorSubcoreMesh(core_axis_name="c", subcore_axis_name="s")

    @functools.partial(
        pl.kernel,
        mesh=mesh,
        out_type=jax.ShapeDtypeStruct((BATCH, 128), jnp.int32),
        scratch_types=[
            pltpu.VMEM((_NCH, _CH), jnp.int32),
            pltpu.VMEM((_BPW, 128), jnp.int32),
            pltpu.SemaphoreType.DMA,
        ],
    )
    def gather_kernel(packed_hbm, idx_hbm, out_hbm, q_v, rows_v, sem):
        p32 = packed_hbm.bitcast(jnp.int32)
        wid = lax.axis_index("s") * _NC + lax.axis_index("c")
        base = wid * _BPW
        for c in range(_NCH):
            pltpu.sync_copy(idx_hbm.at[pl.ds(base + c * _CH, _CH)],
                            q_v.at[c])
        for v in range(_BPW // 16):
            c, o = divmod(v * 16, _CH)
            vec = q_v[c, pl.ds(o, 16)]
            q = jnp.where(vec >= _S, vec - _S, vec)
            q_v[c, pl.ds(o, 16)] = q >> 1
        copies = [
            pltpu.async_copy(
                p32.at[q_v.at[c]],
                rows_v.at[pl.ds(c * _CH, _CH)],
                sem,
            )
            for c in range(_NCH)
        ]
        for cp in copies:
            cp.wait()
        pltpu.sync_copy(rows_v, out_hbm.at[pl.ds(base, _BPW)])

    return gather_kernel(packed, idx)


def _tc_fuse(x_t, gathered, ids_col, W1t, b1c, W2t, b2c, Wf1t, Wf2t, bfc):
    BB = 2048

    def body(xt_ref, g_ref, ids_ref, w1_ref, b1_ref, w2_ref, b2_ref,
             wf1_ref, wf2_ref, bf_ref, out_ref, id_out_ref):
        g = g_ref[...]
        ids = ids_ref[...]
        hi = ids >= _S
        q = jnp.where(hi, ids - _S, ids)
        odd = (q & 1) == 1
        sel = jnp.where(hi, g[:, 64:], g[:, :64])
        bits = jnp.where(odd, sel & jnp.int32(-65536), sel << 16)
        vals = jax.lax.bitcast_convert_type(bits, jnp.float32)
        idt = vals.T
        id_out_ref[...] = idt
        h = jnp.maximum(
            jnp.dot(w1_ref[...], xt_ref[...],
                    preferred_element_type=jnp.float32) + b1_ref[...], 0.0)
        fe = jnp.dot(w2_ref[...], h,
                     preferred_element_type=jnp.float32) + b2_ref[...]
        out_ref[...] = (
            jnp.dot(wf1_ref[...], idt, preferred_element_type=jnp.float32)
            + jnp.dot(wf2_ref[...], fe, preferred_element_type=jnp.float32)
            + bf_ref[...]
        )

    full = lambda i: (0, 0)
    return pl.pallas_call(
        body,
        grid=(BATCH // BB,),
        in_specs=[
            pl.BlockSpec((64, BB), lambda i: (0, i)),
            pl.BlockSpec((BB, 128), lambda i: (i, 0)),
            pl.BlockSpec((BB, 1), lambda i: (i, 0)),
            pl.BlockSpec((64, 64), full),
            pl.BlockSpec((64, 1), full),
            pl.BlockSpec((64, 64), full),
            pl.BlockSpec((64, 1), full),
            pl.BlockSpec((64, 64), full),
            pl.BlockSpec((64, 64), full),
            pl.BlockSpec((64, 1), full),
        ],
        out_specs=[
            pl.BlockSpec((64, BB), lambda i: (0, i)),
            pl.BlockSpec((64, BB), lambda i: (0, i)),
        ],
        out_shape=[
            jax.ShapeDtypeStruct((EMB, BATCH), jnp.float32),
            jax.ShapeDtypeStruct((EMB, BATCH), jnp.float32),
        ],
    )(x_t, gathered, ids_col, W1t, b1c, W2t, b2c, Wf1t, Wf2t, bfc)


def kernel(item_ids, item_features, emb_table, W1, b1, W2, b2, Wf, bf):
    ids = item_ids.astype(jnp.int32)
    table_t = emb_table.T
    x_t = item_features.T
    packed = _tc_pack(table_t)
    gathered = _sc_gather(packed, ids)
    out_t, id_emb_t = _tc_fuse(
        x_t, gathered, ids.reshape(BATCH, 1),
        W1.T, b1.reshape(64, 1), W2.T, b2.reshape(64, 1),
        Wf[:EMB].T, Wf[EMB:].T, bf.reshape(64, 1),
    )
    return out_t.T, id_emb_t.T


# speedup vs baseline: 2.9842x; 1.0452x over previous
import functools

import jax
import jax.numpy as jnp
from jax import lax
from jax.experimental import pallas as pl
from jax.experimental.pallas import tpu as pltpu
from jax.experimental.pallas import tpu_sc as plsc

BATCH = 16384
EMB = 64
NROWS = 1000001

_C = 16384
_SB = 31
_S = _SB * _C
_LASTB = (NROWS + _C - 1) // _C - 1

_NC = 2
_NS = 16
_NW = _NC * _NS
_BPW = BATCH // _NW
_CH = 128
_NCH = _BPW // _CH


def _tc_pack(table_t):
    def body(a_ref, b_ref, out_ref):
        eye = (jax.lax.broadcasted_iota(jnp.int32, (64, 64), 0)
               == jax.lax.broadcasted_iota(jnp.int32, (64, 64), 1)
               ).astype(jnp.float32)
        dn = (((0,), (0,)), ((), ()))
        ta = jax.lax.dot_general(a_ref[...], eye, dn,
                                 preferred_element_type=jnp.float32)
        tb = jax.lax.dot_general(b_ref[...], eye, dn,
                                 preferred_element_type=jnp.float32)
        out_ref[...] = jnp.concatenate([ta, tb], axis=1).astype(jnp.bfloat16)

    return pl.pallas_call(
        body,
        grid=(_SB,),
        in_specs=[
            pl.BlockSpec((64, _C), lambda i: (0, i)),
            pl.BlockSpec((64, _C), lambda i: (0, jnp.minimum(_SB + i, _LASTB))),
        ],
        out_specs=pl.BlockSpec((_C, 128), lambda i: (i, 0)),
        out_shape=jax.ShapeDtypeStruct((_S, 128), jnp.bfloat16),
    )(table_t, table_t)


def _sc_gather(packed, idx):
    mesh = plsc.VectorSubcoreMesh(core_axis_name="c", subcore_axis_name="s")

    @functools.partial(
        pl.kernel,
        mesh=mesh,
        out_type=jax.ShapeDtypeStruct((BATCH, 128), jnp.int32),
        scratch_types=[
            pltpu.VMEM((_NCH, _CH), jnp.int32),
            pltpu.VMEM((_BPW, 128), jnp.int32),
            pltpu.SemaphoreType.DMA,
        ],
    )
    def gather_kernel(packed_hbm, idx_hbm, out_hbm, q_v, rows_v, sem):
        p32 = packed_hbm.bitcast(jnp.int32)
        wid = lax.axis_index("s") * _NC + lax.axis_index("c")
        base = wid * _BPW
        for c in range(_NCH):
            pltpu.sync_copy(idx_hbm.at[pl.ds(base + c * _CH, _CH)],
                            q_v.at[c])
        for v in range(_BPW // 16):
            c, o = divmod(v * 16, _CH)
            vec = q_v[c, pl.ds(o, 16)]
            q = jnp.where(vec >= _S, vec - _S, vec)
            q_v[c, pl.ds(o, 16)] = q >> 1
        copies = [
            pltpu.async_copy(
                p32.at[q_v.at[c]],
                rows_v.at[pl.ds(c * _CH, _CH)],
                sem,
            )
            for c in range(_NCH)
        ]
        for cp in copies:
            cp.wait()
        pltpu.sync_copy(rows_v, out_hbm.at[pl.ds(base, _BPW)])

    return gather_kernel(packed, idx)


def _tc_fuse(x_t, gathered, ids_col, W1t, b1c, W2t, b2c, Wf1t, Wf2t, bfc):
    BB = 2048

    def body(xt_ref, g_ref, ids_ref, w1_ref, b1_ref, w2_ref, b2_ref,
             wf1_ref, wf2_ref, bf_ref, out_ref, id_out_ref):
        g = g_ref[...]
        ids = ids_ref[...]
        hi = ids >= _S
        q = jnp.where(hi, ids - _S, ids)
        odd = (q & 1) == 1
        sel = jnp.where(hi, g[:, 64:], g[:, :64])
        bits = jnp.where(odd, sel & jnp.int32(-65536), sel << 16)
        vals = jax.lax.bitcast_convert_type(bits, jnp.float32)
        idt = vals.T
        id_out_ref[...] = idt
        h = jnp.maximum(
            jnp.dot(w1_ref[...], xt_ref[...],
                    preferred_element_type=jnp.float32) + b1_ref[...], 0.0)
        fe = jnp.dot(w2_ref[...], h,
                     preferred_element_type=jnp.float32) + b2_ref[...]
        out_ref[...] = (
            jnp.dot(wf1_ref[...], idt, preferred_element_type=jnp.float32)
            + jnp.dot(wf2_ref[...], fe, preferred_element_type=jnp.float32)
            + bf_ref[...]
        )

    full = lambda i: (0, 0)
    return pl.pallas_call(
        body,
        grid=(BATCH // BB,),
        in_specs=[
            pl.BlockSpec((64, BB), lambda i: (0, i)),
            pl.BlockSpec((BB, 128), lambda i: (i, 0)),
            pl.BlockSpec((BB, 1), lambda i: (i, 0)),
            pl.BlockSpec((64, 64), full),
            pl.BlockSpec((64, 1), full),
            pl.BlockSpec((64, 64), full),
            pl.BlockSpec((64, 1), full),
            pl.BlockSpec((64, 64), full),
            pl.BlockSpec((64, 64), full),
            pl.BlockSpec((64, 1), full),
        ],
        out_specs=[
            pl.BlockSpec((64, BB), lambda i: (0, i)),
            pl.BlockSpec((64, BB), lambda i: (0, i)),
        ],
        out_shape=[
            jax.ShapeDtypeStruct((EMB, BATCH), jnp.float32),
            jax.ShapeDtypeStruct((EMB, BATCH), jnp.float32),
        ],
    )(x_t, gathered, ids_col, W1t, b1c, W2t, b2c, Wf1t, Wf2t, bfc)


def kernel(item_ids, item_features, emb_table, W1, b1, W2, b2, Wf, bf):
    ids = item_ids.astype(jnp.int32)
    table_t = emb_table.T
    x_t = item_features.T
    packed = _tc_pack(table_t)
    gathered = _sc_gather(packed, ids)
    out_t, id_emb_t = _tc_fuse(
        x_t, gathered, ids.reshape(BATCH, 1),
        W1.T, b1.reshape(64, 1), W2.T, b2.reshape(64, 1),
        Wf[:EMB].T, Wf[EMB:].T, bf.reshape(64, 1),
    )
    return out_t.T, id_emb_t.T


# speedup vs baseline: 2.9941x; 1.0033x over previous
import functools

import jax
import jax.numpy as jnp
from jax import lax
from jax.experimental import pallas as pl
from jax.experimental.pallas import tpu as pltpu
from jax.experimental.pallas import tpu_sc as plsc

BATCH = 16384
EMB = 64
NROWS = 1000001

_C = 16384
_SB = 31
_S = _SB * _C
_LASTB = (NROWS + _C - 1) // _C - 1

_NC = 2
_NS = 16
_NW = _NC * _NS
_BPW = BATCH // _NW
_CH = 128
_NCH = _BPW // _CH


def _tc_pack(table_t):
    def body(a_ref, b_ref, out_ref):
        eye = (jax.lax.broadcasted_iota(jnp.int32, (64, 64), 0)
               == jax.lax.broadcasted_iota(jnp.int32, (64, 64), 1)
               ).astype(jnp.float32)
        dn = (((0,), (0,)), ((), ()))
        ta = jax.lax.dot_general(a_ref[...], eye, dn,
                                 preferred_element_type=jnp.float32)
        tb = jax.lax.dot_general(b_ref[...], eye, dn,
                                 preferred_element_type=jnp.float32)
        out_ref[...] = jnp.concatenate([ta, tb], axis=1).astype(jnp.bfloat16)

    return pl.pallas_call(
        body,
        grid=(_SB,),
        in_specs=[
            pl.BlockSpec((64, _C), lambda i: (0, i)),
            pl.BlockSpec((64, _C), lambda i: (0, jnp.minimum(_SB + i, _LASTB))),
        ],
        out_specs=pl.BlockSpec((_C, 128), lambda i: (i, 0)),
        out_shape=jax.ShapeDtypeStruct((_S, 128), jnp.bfloat16),
    )(table_t, table_t)


def _sc_gather(packed, idx):
    mesh = plsc.VectorSubcoreMesh(core_axis_name="c", subcore_axis_name="s")

    @functools.partial(
        pl.kernel,
        mesh=mesh,
        out_type=jax.ShapeDtypeStruct((BATCH, 128), jnp.int32),
        scratch_types=[
            pltpu.VMEM((_NCH, _CH), jnp.int32),
            pltpu.VMEM((_BPW, 128), jnp.int32),
            pltpu.SemaphoreType.DMA,
        ],
    )
    def gather_kernel(packed_hbm, idx_hbm, out_hbm, q_v, rows_v, sem):
        p32 = packed_hbm.bitcast(jnp.int32)
        wid = lax.axis_index("s") * _NC + lax.axis_index("c")
        base = wid * _BPW
        for c in range(_NCH):
            pltpu.sync_copy(idx_hbm.at[pl.ds(base + c * _CH, _CH)],
                            q_v.at[c])
        for v in range(_BPW // 16):
            c, o = divmod(v * 16, _CH)
            vec = q_v[c, pl.ds(o, 16)]
            q = jnp.where(vec >= _S, vec - _S, vec)
            q_v[c, pl.ds(o, 16)] = q >> 1
        copies = [
            pltpu.async_copy(
                p32.at[q_v.at[c]],
                rows_v.at[pl.ds(c * _CH, _CH)],
                sem,
            )
            for c in range(_NCH)
        ]
        for cp in copies:
            cp.wait()
        pltpu.sync_copy(rows_v, out_hbm.at[pl.ds(base, _BPW)])

    return gather_kernel(packed, idx)


def _tc_fuse(x_t, gathered, ids_col, W1t, b1c, W2t, b2c, Wf1t, Wf2t, bfc):
    BB = 4096

    def body(xt_ref, g_ref, ids_ref, w1_ref, b1_ref, w2_ref, b2_ref,
             wf1_ref, wf2_ref, bf_ref, out_ref, id_out_ref):
        g = g_ref[...]
        ids = ids_ref[...]
        hi = ids >= _S
        q = jnp.where(hi, ids - _S, ids)
        odd = (q & 1) == 1
        sel = jnp.where(hi, g[:, 64:], g[:, :64])
        bits = jnp.where(odd, sel & jnp.int32(-65536), sel << 16)
        vals = jax.lax.bitcast_convert_type(bits, jnp.float32)
        idt = vals.T
        id_out_ref[...] = idt
        h = jnp.maximum(
            jnp.dot(w1_ref[...], xt_ref[...],
                    preferred_element_type=jnp.float32) + b1_ref[...], 0.0)
        fe = jnp.dot(w2_ref[...], h,
                     preferred_element_type=jnp.float32) + b2_ref[...]
        out_ref[...] = (
            jnp.dot(wf1_ref[...], idt, preferred_element_type=jnp.float32)
            + jnp.dot(wf2_ref[...], fe, preferred_element_type=jnp.float32)
            + bf_ref[...]
        )

    full = lambda i: (0, 0)
    return pl.pallas_call(
        body,
        grid=(BATCH // BB,),
        in_specs=[
            pl.BlockSpec((64, BB), lambda i: (0, i)),
            pl.BlockSpec((BB, 128), lambda i: (i, 0)),
            pl.BlockSpec((BB, 1), lambda i: (i, 0)),
            pl.BlockSpec((64, 64), full),
            pl.BlockSpec((64, 1), full),
            pl.BlockSpec((64, 64), full),
            pl.BlockSpec((64, 1), full),
            pl.BlockSpec((64, 64), full),
            pl.BlockSpec((64, 64), full),
            pl.BlockSpec((64, 1), full),
        ],
        out_specs=[
            pl.BlockSpec((64, BB), lambda i: (0, i)),
            pl.BlockSpec((64, BB), lambda i: (0, i)),
        ],
        out_shape=[
            jax.ShapeDtypeStruct((EMB, BATCH), jnp.float32),
            jax.ShapeDtypeStruct((EMB, BATCH), jnp.float32),
        ],
    )(x_t, gathered, ids_col, W1t, b1c, W2t, b2c, Wf1t, Wf2t, bfc)


def kernel(item_ids, item_features, emb_table, W1, b1, W2, b2, Wf, bf):
    ids = item_ids.astype(jnp.int32)
    table_t = emb_table.T
    x_t = item_features.T
    packed = _tc_pack(table_t)
    gathered = _sc_gather(packed, ids)
    out_t, id_emb_t = _tc_fuse(
        x_t, gathered, ids.reshape(BATCH, 1),
        W1.T, b1.reshape(64, 1), W2.T, b2.reshape(64, 1),
        Wf[:EMB].T, Wf[EMB:].T, bf.reshape(64, 1),
    )
    return out_t.T, id_emb_t.T
